# baseline (speedup 1.0000x reference)
import functools

import jax
import jax.numpy as jnp
from jax import lax
from jax.experimental import pallas as pl
from jax.experimental.pallas import tpu as pltpu
from jax.experimental.pallas import tpu_sc as plsc

B, C, N = 8, 6, 16384
H = 256
NKEY = 256

MBLK = 2048
NB = N // MBLK

_R = B * C
_DPAD = 128
_IMIN = -2147483648

_ST = 16
_SLICE = N // _ST
_NCH = _SLICE // 16
_LCAP = N + _ST * 16
_OPAD = NKEY + 16


def _mlp_keys_body(x_ref, w1t_ref, b1_ref, w2t_ref, b2_ref, w3r_ref,
                   keys_ref, t_ref, tbl_ref, scores_ref):
    pid = pl.program_id(0)
    w1t = w1t_ref[...]
    b1 = b1_ref[...]
    w2t = w2t_ref[...]
    b2 = b2_ref[...]
    w3r = w3r_ref[...]
    acc = jnp.zeros((1, MBLK), jnp.float32)
    for b in range(B):
        x = x_ref[b]
        h = jax.lax.dot_general(w1t, x, (((1,), (0,)), ((), ())),
                                preferred_element_type=jnp.float32)
        h = jnp.maximum(h + b1, 0.0)
        f = jax.lax.dot_general(w2t, h, (((1,), (0,)), ((), ())),
                                preferred_element_type=jnp.float32)
        f = jnp.maximum(f + b2, 0.0)
        s = jax.lax.dot_general(w3r, f, (((1,), (0,)), ((), ())),
                                preferred_element_type=jnp.float32)
        acc = acc + s

    xall = x_ref[...].reshape(_R, MBLK)
    t = jnp.transpose(xall, (1, 0))
    tbl_ref[...] = jnp.concatenate(
        [t, jnp.zeros((MBLK, _DPAD - _R), jnp.float32)], axis=1)

    scores_ref[pl.ds(pid, 1), :] = acc

    @pl.when(pid == NB - 1)
    def _():
        a = scores_ref[...]
        bk = jax.lax.bitcast_convert_type(a, jnp.int32)
        keys = jnp.where(bk >= 0, bk, bk ^ jnp.int32(0x7FFFFFFF))
        keys_ref[...] = keys

        cnt0 = jnp.sum(jnp.where(keys >= 0, 1.0, 0.0))
        tinit = jnp.where(cnt0 >= float(NKEY), jnp.int32(0), jnp.int32(_IMIN))

        def sbody(k, t0):
            cand = t0 + (jnp.int32(1) << (30 - k))
            cnt = jnp.sum(jnp.where(keys >= cand, 1.0, 0.0))
            return jnp.where(cnt >= float(NKEY), cand, t0)

        thr = lax.fori_loop(0, 31, sbody, tinit)
        t_ref[...] = jnp.full((1, 128), thr, jnp.int32)


def _mlp_keys(src_pts, W1, b1, W2, b2, W3):
    w1t = jnp.transpose(W1)
    w2t = jnp.transpose(W2)
    w3r = jnp.transpose(W3)
    b1c = b1[:, None]
    b2c = b2[:, None]
    return pl.pallas_call(
        _mlp_keys_body,
        grid=(NB,),
        in_specs=[
            pl.BlockSpec((B, C, MBLK), lambda i: (0, 0, i)),
            pl.BlockSpec((H, C), lambda i: (0, 0)),
            pl.BlockSpec((H, 1), lambda i: (0, 0)),
            pl.BlockSpec((H, H), lambda i: (0, 0)),
            pl.BlockSpec((H, 1), lambda i: (0, 0)),
            pl.BlockSpec((1, H), lambda i: (0, 0)),
        ],
        out_specs=[
            pl.BlockSpec((NB, MBLK), lambda i: (0, 0)),
            pl.BlockSpec((1, 128), lambda i: (0, 0)),
            pl.BlockSpec((MBLK, _DPAD), lambda i: (i, 0)),
        ],
        out_shape=[
            jax.ShapeDtypeStruct((NB, MBLK), jnp.int32),
            jax.ShapeDtypeStruct((1, 128), jnp.int32),
            jax.ShapeDtypeStruct((N, _DPAD), jnp.float32),
        ],
        scratch_shapes=[pltpu.VMEM((NB, MBLK), jnp.float32)],
    )(src_pts, w1t, b1c, w2t, b2c, w3r)


def _sc_body(tbl_hbm, keys_hbm, t_hbm, out_hbm,
             kv, tv, bufk, bufi, bufp, cntb, lk, li, callv, idxt, post,
             shk, shi, cnts_sh, rows_v, sem):
    wid = lax.axis_index("s")
    base = wid * _SLICE
    iota16 = lax.iota(jnp.int32, 16)
    zero16 = jnp.zeros((16,), jnp.int32)

    pltpu.sync_copy(keys_hbm.at[pl.ds(base, _SLICE)], kv)
    pltpu.sync_copy(t_hbm.at[pl.ds(0, 16)], tv)
    thr = tv[...]

    def ibody(j, _):
        bufk[pl.ds(j * 16, 16)] = zero16 + _IMIN
        bufi[pl.ds(j * 16, 16)] = zero16
        bufp[pl.ds(j * 16, 16)] = zero16 + NKEY
        return 0
    lax.fori_loop(0, _NCH, ibody, 0)

    def cbody(j, cnt):
        k16 = kv[pl.ds(j * 16, 16)]
        m = k16 >= thr
        mi = jnp.where(m, 1, 0)
        pos = cnt + plsc.cumsum(mi) - 1
        g16 = base + j * 16 + iota16
        plsc.store_scatter(bufk, [pos], k16, mask=m)
        plsc.store_scatter(bufi, [pos], g16, mask=m)
        return cnt + jnp.sum(mi)
    cnt = lax.fori_loop(0, _NCH, cbody, jnp.int32(0))

    cntb[...] = zero16 + cnt
    pltpu.sync_copy(cntb, cnts_sh.at[pl.ds(wid * 16, 16)])
    pltpu.sync_copy(bufk, shk.at[pl.ds(wid * _SLICE, _SLICE)])
    pltpu.sync_copy(bufi, shi.at[pl.ds(wid * _SLICE, _SLICE)])
    plsc.subcore_barrier()
    pltpu.sync_copy(cnts_sh, callv)

    totch = jnp.int32(0)
    for t in range(_ST):
        cnt_t = jnp.max(callv[pl.ds(t * 16, 16)])

        def cstage(c2, off2, t=t):
            src = t * _SLICE + c2 * 16
            pltpu.sync_copy(shk.at[pl.ds(src, 16)], lk.at[pl.ds(off2 * 16, 16)])
            pltpu.sync_copy(shi.at[pl.ds(src, 16)], li.at[pl.ds(off2 * 16, 16)])
            return off2 + 1
        totch = lax.fori_loop(0, (cnt_t + 15) // 16, cstage, totch)

    def rbody(c, _):
        ch = c // 16
        ln = c % 16
        onel = iota16 == ln
        kc = jnp.sum(jnp.where(onel, bufk[pl.ds(ch * 16, 16)], 0))
        gc = jnp.sum(jnp.where(onel, bufi[pl.ds(ch * 16, 16)], 0))

        def sscan(j, accv):
            ko = lk[pl.ds(j * 16, 16)]
            go = li[pl.ds(j * 16, 16)]
            beat = (ko > kc) | ((ko == kc) & (go < gc))
            return accv + jnp.where(beat, 1, 0)
        accv = lax.fori_loop(0, totch, sscan, zero16)
        pos = jnp.minimum(jnp.sum(accv), NKEY)
        plsc.store_scatter(bufp, [zero16 + c], zero16 + pos, mask=onel)
        return 0
    lax.fori_loop(0, cnt, rbody, 0)

    def dbody(j, _):
        idxt[...] = bufi[pl.ds(j * 16, 16)]
        post[...] = bufp[pl.ds(j * 16, 16)]
        pltpu.async_copy(tbl_hbm.at[idxt], rows_v, sem).wait()
        pltpu.async_copy(rows_v, out_hbm.at[post], sem).wait()
        return 0
    lax.fori_loop(0, (cnt + 15) // 16, dbody, 0)


@functools.cache
def _sc_topk_gather():
    return pl.kernel(
        _sc_body,
        mesh=plsc.VectorSubcoreMesh(core_axis_name="c", subcore_axis_name="s",
                                    num_cores=1),
        out_type=jax.ShapeDtypeStruct((_OPAD, _DPAD), jnp.float32),
        scratch_types=[
            pltpu.VMEM((_SLICE,), jnp.int32),
            pltpu.VMEM((16,), jnp.int32),
            pltpu.VMEM((_SLICE,), jnp.int32),
            pltpu.VMEM((_SLICE,), jnp.int32),
            pltpu.VMEM((_SLICE,), jnp.int32),
            pltpu.VMEM((16,), jnp.int32),
            pltpu.VMEM((_LCAP,), jnp.int32),
            pltpu.VMEM((_LCAP,), jnp.int32),
            pltpu.VMEM((_ST * 16,), jnp.int32),
            pltpu.VMEM((16,), jnp.int32),
            pltpu.VMEM((16,), jnp.int32),
            pltpu.VMEM_SHARED((_ST * _SLICE,), jnp.int32),
            pltpu.VMEM_SHARED((_ST * _SLICE,), jnp.int32),
            pltpu.VMEM_SHARED((_ST * 16,), jnp.int32),
            pltpu.VMEM((16, _DPAD), jnp.float32),
            pltpu.SemaphoreType.DMA,
        ],
        compiler_params=pltpu.CompilerParams(needs_layout_passes=False),
    )


def kernel(src_pts, tgt_pts, W1, b1, W2, b2, W3, b3):
    keys2d, thr2d, tbl = _mlp_keys(src_pts, W1, b1, W2, b2, W3)
    keys = keys2d.reshape(N)
    thr = thr2d.reshape(128)
    out = _sc_topk_gather()(tbl, keys, thr)
    g = out[:NKEY, :_R]
    return jnp.transpose(g.reshape(NKEY, B, C), (1, 0, 2))


# speedup vs baseline: 1.0073x; 1.0073x over previous
import functools

import jax
import jax.numpy as jnp
from jax import lax
from jax.experimental import pallas as pl
from jax.experimental.pallas import tpu as pltpu
from jax.experimental.pallas import tpu_sc as plsc

B, C, N = 8, 6, 16384
H = 256
NKEY = 256

MBLK = 4096
NB = N // MBLK

_R = B * C
_DPAD = 128
_IMIN = -2147483648

_ST = 16
_SLICE = N // _ST
_NCH = _SLICE // 16
_LCAP = N + _ST * 16
_OPAD = NKEY + 16


def _mlp_keys_body(x_ref, w1t_ref, b1_ref, w2t_ref, b2_ref, w3r_ref,
                   keys_ref, t_ref, tbl_ref, scores_ref):
    pid = pl.program_id(0)
    w1t = w1t_ref[...]
    b1 = b1_ref[...]
    w2t = w2t_ref[...]
    b2 = b2_ref[...]
    w3r = w3r_ref[...]
    acc = jnp.zeros((1, MBLK), jnp.float32)
    for b in range(B):
        x = x_ref[b]
        h = jax.lax.dot_general(w1t, x, (((1,), (0,)), ((), ())),
                                preferred_element_type=jnp.float32)
        h = jnp.maximum(h + b1, 0.0)
        f = jax.lax.dot_general(w2t, h, (((1,), (0,)), ((), ())),
                                preferred_element_type=jnp.float32)
        f = jnp.maximum(f + b2, 0.0)
        s = jax.lax.dot_general(w3r, f, (((1,), (0,)), ((), ())),
                                preferred_element_type=jnp.float32)
        acc = acc + s

    xall = x_ref[...].reshape(_R, MBLK)
    t = jnp.transpose(xall, (1, 0))
    tbl_ref[...] = jnp.concatenate(
        [t, jnp.zeros((MBLK, _DPAD - _R), jnp.float32)], axis=1)

    scores_ref[pl.ds(pid, 1), :] = acc

    @pl.when(pid == NB - 1)
    def _():
        a = scores_ref[...]
        bk = jax.lax.bitcast_convert_type(a, jnp.int32)
        keys = jnp.where(bk >= 0, bk, bk ^ jnp.int32(0x7FFFFFFF))
        keys_ref[...] = keys

        cnt0 = jnp.sum(jnp.where(keys >= 0, 1.0, 0.0))
        tinit = jnp.where(cnt0 >= float(NKEY), jnp.int32(0), jnp.int32(_IMIN))

        def sbody(k, t0):
            cand = t0 + (jnp.int32(1) << (30 - k))
            cnt = jnp.sum(jnp.where(keys >= cand, 1.0, 0.0))
            return jnp.where(cnt >= float(NKEY), cand, t0)

        thr = lax.fori_loop(0, 31, sbody, tinit)
        t_ref[...] = jnp.full((1, 128), thr, jnp.int32)


def _mlp_keys(src_pts, W1, b1, W2, b2, W3):
    w1t = jnp.transpose(W1)
    w2t = jnp.transpose(W2)
    w3r = jnp.transpose(W3)
    b1c = b1[:, None]
    b2c = b2[:, None]
    return pl.pallas_call(
        _mlp_keys_body,
        grid=(NB,),
        in_specs=[
            pl.BlockSpec((B, C, MBLK), lambda i: (0, 0, i)),
            pl.BlockSpec((H, C), lambda i: (0, 0)),
            pl.BlockSpec((H, 1), lambda i: (0, 0)),
            pl.BlockSpec((H, H), lambda i: (0, 0)),
            pl.BlockSpec((H, 1), lambda i: (0, 0)),
            pl.BlockSpec((1, H), lambda i: (0, 0)),
        ],
        out_specs=[
            pl.BlockSpec((NB, MBLK), lambda i: (0, 0)),
            pl.BlockSpec((1, 128), lambda i: (0, 0)),
            pl.BlockSpec((MBLK, _DPAD), lambda i: (i, 0)),
        ],
        out_shape=[
            jax.ShapeDtypeStruct((NB, MBLK), jnp.int32),
            jax.ShapeDtypeStruct((1, 128), jnp.int32),
            jax.ShapeDtypeStruct((N, _DPAD), jnp.float32),
        ],
        scratch_shapes=[pltpu.VMEM((NB, MBLK), jnp.float32)],
    )(src_pts, w1t, b1c, w2t, b2c, w3r)


def _sc_body(tbl_hbm, keys_hbm, t_hbm, out_hbm,
             kv, tv, bufk, bufi, bufp, cntb, lk, li, callv, idxt, post,
             shk, shi, cnts_sh, rows_v, sem):
    wid = lax.axis_index("s")
    base = wid * _SLICE
    iota16 = lax.iota(jnp.int32, 16)
    zero16 = jnp.zeros((16,), jnp.int32)

    pltpu.sync_copy(keys_hbm.at[pl.ds(base, _SLICE)], kv)
    pltpu.sync_copy(t_hbm.at[pl.ds(0, 16)], tv)
    thr = tv[...]

    def ibody(j, _):
        bufk[pl.ds(j * 16, 16)] = zero16 + _IMIN
        bufi[pl.ds(j * 16, 16)] = zero16
        bufp[pl.ds(j * 16, 16)] = zero16 + NKEY
        return 0
    lax.fori_loop(0, _NCH, ibody, 0)

    def cbody(j, cnt):
        k16 = kv[pl.ds(j * 16, 16)]
        m = k16 >= thr
        mi = jnp.where(m, 1, 0)
        pos = cnt + plsc.cumsum(mi) - 1
        g16 = base + j * 16 + iota16
        plsc.store_scatter(bufk, [pos], k16, mask=m)
        plsc.store_scatter(bufi, [pos], g16, mask=m)
        return cnt + jnp.sum(mi)
    cnt = lax.fori_loop(0, _NCH, cbody, jnp.int32(0))

    cntb[...] = zero16 + cnt
    pltpu.sync_copy(cntb, cnts_sh.at[pl.ds(wid * 16, 16)])
    pltpu.sync_copy(bufk, shk.at[pl.ds(wid * _SLICE, _SLICE)])
    pltpu.sync_copy(bufi, shi.at[pl.ds(wid * _SLICE, _SLICE)])
    plsc.subcore_barrier()
    pltpu.sync_copy(cnts_sh, callv)

    totch = jnp.int32(0)
    for t in range(_ST):
        cnt_t = jnp.max(callv[pl.ds(t * 16, 16)])

        def cstage(c2, off2, t=t):
            src = t * _SLICE + c2 * 16
            pltpu.sync_copy(shk.at[pl.ds(src, 16)], lk.at[pl.ds(off2 * 16, 16)])
            pltpu.sync_copy(shi.at[pl.ds(src, 16)], li.at[pl.ds(off2 * 16, 16)])
            return off2 + 1
        totch = lax.fori_loop(0, (cnt_t + 15) // 16, cstage, totch)

    def rbody(c, _):
        ch = c // 16
        ln = c % 16
        onel = iota16 == ln
        kc = jnp.sum(jnp.where(onel, bufk[pl.ds(ch * 16, 16)], 0))
        gc = jnp.sum(jnp.where(onel, bufi[pl.ds(ch * 16, 16)], 0))

        def sscan(j, accv):
            ko = lk[pl.ds(j * 16, 16)]
            go = li[pl.ds(j * 16, 16)]
            beat = (ko > kc) | ((ko == kc) & (go < gc))
            return accv + jnp.where(beat, 1, 0)
        accv = lax.fori_loop(0, totch, sscan, zero16)
        pos = jnp.minimum(jnp.sum(accv), NKEY)
        plsc.store_scatter(bufp, [zero16 + c], zero16 + pos, mask=onel)
        return 0
    lax.fori_loop(0, cnt, rbody, 0)

    def dbody(j, _):
        idxt[...] = bufi[pl.ds(j * 16, 16)]
        post[...] = bufp[pl.ds(j * 16, 16)]
        pltpu.async_copy(tbl_hbm.at[idxt], rows_v, sem).wait()
        pltpu.async_copy(rows_v, out_hbm.at[post], sem).wait()
        return 0
    lax.fori_loop(0, (cnt + 15) // 16, dbody, 0)


@functools.cache
def _sc_topk_gather():
    return pl.kernel(
        _sc_body,
        mesh=plsc.VectorSubcoreMesh(core_axis_name="c", subcore_axis_name="s",
                                    num_cores=1),
        out_type=jax.ShapeDtypeStruct((_OPAD, _DPAD), jnp.float32),
        scratch_types=[
            pltpu.VMEM((_SLICE,), jnp.int32),
            pltpu.VMEM((16,), jnp.int32),
            pltpu.VMEM((_SLICE,), jnp.int32),
            pltpu.VMEM((_SLICE,), jnp.int32),
            pltpu.VMEM((_SLICE,), jnp.int32),
            pltpu.VMEM((16,), jnp.int32),
            pltpu.VMEM((_LCAP,), jnp.int32),
            pltpu.VMEM((_LCAP,), jnp.int32),
            pltpu.VMEM((_ST * 16,), jnp.int32),
            pltpu.VMEM((16,), jnp.int32),
            pltpu.VMEM((16,), jnp.int32),
            pltpu.VMEM_SHARED((_ST * _SLICE,), jnp.int32),
            pltpu.VMEM_SHARED((_ST * _SLICE,), jnp.int32),
            pltpu.VMEM_SHARED((_ST * 16,), jnp.int32),
            pltpu.VMEM((16, _DPAD), jnp.float32),
            pltpu.SemaphoreType.DMA,
        ],
        compiler_params=pltpu.CompilerParams(needs_layout_passes=False),
    )


def kernel(src_pts, tgt_pts, W1, b1, W2, b2, W3, b3):
    keys2d, thr2d, tbl = _mlp_keys(src_pts, W1, b1, W2, b2, W3)
    keys = keys2d.reshape(N)
    thr = thr2d.reshape(128)
    out = _sc_topk_gather()(tbl, keys, thr)
    g = out[:NKEY, :_R]
    return jnp.transpose(g.reshape(NKEY, B, C), (1, 0, 2))


# speedup vs baseline: 1.0248x; 1.0174x over previous
import functools

import jax
import jax.numpy as jnp
from jax import lax
from jax.experimental import pallas as pl
from jax.experimental.pallas import tpu as pltpu
from jax.experimental.pallas import tpu_sc as plsc

B, C, N = 8, 6, 16384
H = 256
NKEY = 256

MBLK = 4096
NB = N // MBLK

_R = B * C
_DPAD = 128
_IMIN = -2147483648

_ST = 16
_SLICE = N // _ST
_NCH = _SLICE // 16
_LCAP = N + _ST * 16
_OPAD = NKEY + 16


def _mlp_keys_body(x_ref, w1_ref, b1_ref, w2_ref, b2_ref, w3_ref,
                   keys_ref, t_ref, tbl_ref, scores_ref):
    pid = pl.program_id(0)
    w1 = w1_ref[...]
    b1 = b1_ref[...]
    w2 = w2_ref[...]
    b2 = b2_ref[...]
    w3 = w3_ref[...]
    acc = jnp.zeros((1, MBLK), jnp.float32)
    for b in range(B):
        x = x_ref[b]
        h = jax.lax.dot_general(w1, x, (((0,), (0,)), ((), ())),
                                preferred_element_type=jnp.float32)
        h = jnp.maximum(h + b1, 0.0)
        f = jax.lax.dot_general(w2, h, (((0,), (0,)), ((), ())),
                                preferred_element_type=jnp.float32)
        f = jnp.maximum(f + b2, 0.0)
        s = jax.lax.dot_general(w3, f, (((0,), (0,)), ((), ())),
                                preferred_element_type=jnp.float32)
        acc = acc + s

    xall = x_ref[...].reshape(_R, MBLK)
    t = jnp.transpose(xall, (1, 0))
    tbl_ref[...] = jnp.concatenate(
        [t, jnp.zeros((MBLK, _DPAD - _R), jnp.float32)], axis=1)

    scores_ref[pl.ds(pid, 1), :] = acc

    @pl.when(pid == NB - 1)
    def _():
        a = scores_ref[...]
        bk = jax.lax.bitcast_convert_type(a, jnp.int32)
        keys = jnp.where(bk >= 0, bk, bk ^ jnp.int32(0x7FFFFFFF))
        keys_ref[...] = keys

        cnt0 = jnp.sum(jnp.where(keys >= 0, 1.0, 0.0))
        tinit = jnp.where(cnt0 >= float(NKEY), jnp.int32(0), jnp.int32(_IMIN))

        def sbody(k, t0):
            cand = t0 + (jnp.int32(1) << (30 - k))
            cnt = jnp.sum(jnp.where(keys >= cand, 1.0, 0.0))
            return jnp.where(cnt >= float(NKEY), cand, t0)

        thr = lax.fori_loop(0, 31, sbody, tinit)
        t_ref[...] = jnp.full((1, 128), thr, jnp.int32)


def _mlp_keys(src_pts, W1, b1, W2, b2, W3):
    b1c = b1[:, None]
    b2c = b2[:, None]
    return pl.pallas_call(
        _mlp_keys_body,
        grid=(NB,),
        in_specs=[
            pl.BlockSpec((B, C, MBLK), lambda i: (0, 0, i)),
            pl.BlockSpec((C, H), lambda i: (0, 0)),
            pl.BlockSpec((H, 1), lambda i: (0, 0)),
            pl.BlockSpec((H, H), lambda i: (0, 0)),
            pl.BlockSpec((H, 1), lambda i: (0, 0)),
            pl.BlockSpec((H, 1), lambda i: (0, 0)),
        ],
        out_specs=[
            pl.BlockSpec((NB, MBLK), lambda i: (0, 0)),
            pl.BlockSpec((1, 128), lambda i: (0, 0)),
            pl.BlockSpec((MBLK, _DPAD), lambda i: (i, 0)),
        ],
        out_shape=[
            jax.ShapeDtypeStruct((NB, MBLK), jnp.int32),
            jax.ShapeDtypeStruct((1, 128), jnp.int32),
            jax.ShapeDtypeStruct((N, _DPAD), jnp.float32),
        ],
        scratch_shapes=[pltpu.VMEM((NB, MBLK), jnp.float32)],
    )(src_pts, W1, b1c, W2, b2c, W3)


def _sc_body(tbl_hbm, keys_hbm, t_hbm, out_hbm,
             kv, tv, bufk, bufi, bufp, cntb, lk, li, callv, idxt, post,
             shk, shi, cnts_sh, rows_v, sem):
    wid = lax.axis_index("s")
    base = wid * _SLICE
    iota16 = lax.iota(jnp.int32, 16)
    zero16 = jnp.zeros((16,), jnp.int32)

    pltpu.sync_copy(keys_hbm.at[pl.ds(base, _SLICE)], kv)
    pltpu.sync_copy(t_hbm.at[pl.ds(0, 16)], tv)
    thr = tv[...]

    def ibody(j, _):
        bufk[pl.ds(j * 16, 16)] = zero16 + _IMIN
        bufi[pl.ds(j * 16, 16)] = zero16
        bufp[pl.ds(j * 16, 16)] = zero16 + NKEY
        return 0
    lax.fori_loop(0, _NCH, ibody, 0)

    def cbody(j, cnt):
        k16 = kv[pl.ds(j * 16, 16)]
        m = k16 >= thr
        mi = jnp.where(m, 1, 0)
        pos = cnt + plsc.cumsum(mi) - 1
        g16 = base + j * 16 + iota16
        plsc.store_scatter(bufk, [pos], k16, mask=m)
        plsc.store_scatter(bufi, [pos], g16, mask=m)
        return cnt + jnp.sum(mi)
    cnt = lax.fori_loop(0, _NCH, cbody, jnp.int32(0))

    cntb[...] = zero16 + cnt
    pltpu.sync_copy(cntb, cnts_sh.at[pl.ds(wid * 16, 16)])
    pltpu.sync_copy(bufk, shk.at[pl.ds(wid * _SLICE, _SLICE)])
    pltpu.sync_copy(bufi, shi.at[pl.ds(wid * _SLICE, _SLICE)])
    plsc.subcore_barrier()
    pltpu.sync_copy(cnts_sh, callv)

    totch = jnp.int32(0)
    for t in range(_ST):
        cnt_t = jnp.max(callv[pl.ds(t * 16, 16)])

        def cstage(c2, off2, t=t):
            src = t * _SLICE + c2 * 16
            pltpu.sync_copy(shk.at[pl.ds(src, 16)], lk.at[pl.ds(off2 * 16, 16)])
            pltpu.sync_copy(shi.at[pl.ds(src, 16)], li.at[pl.ds(off2 * 16, 16)])
            return off2 + 1
        totch = lax.fori_loop(0, (cnt_t + 15) // 16, cstage, totch)

    def rbody(c, _):
        ch = c // 16
        ln = c % 16
        onel = iota16 == ln
        kc = jnp.sum(jnp.where(onel, bufk[pl.ds(ch * 16, 16)], 0))
        gc = jnp.sum(jnp.where(onel, bufi[pl.ds(ch * 16, 16)], 0))

        def sscan(j, accv):
            ko = lk[pl.ds(j * 16, 16)]
            go = li[pl.ds(j * 16, 16)]
            beat = (ko > kc) | ((ko == kc) & (go < gc))
            return accv + jnp.where(beat, 1, 0)
        accv = lax.fori_loop(0, totch, sscan, zero16)
        pos = jnp.minimum(jnp.sum(accv), NKEY)
        plsc.store_scatter(bufp, [zero16 + c], zero16 + pos, mask=onel)
        return 0
    lax.fori_loop(0, cnt, rbody, 0)

    def dbody(j, _):
        idxt[...] = bufi[pl.ds(j * 16, 16)]
        post[...] = bufp[pl.ds(j * 16, 16)]
        pltpu.async_copy(tbl_hbm.at[idxt], rows_v, sem).wait()
        pltpu.async_copy(rows_v, out_hbm.at[post], sem).wait()
        return 0
    lax.fori_loop(0, (cnt + 15) // 16, dbody, 0)


@functools.cache
def _sc_topk_gather():
    return pl.kernel(
        _sc_body,
        mesh=plsc.VectorSubcoreMesh(core_axis_name="c", subcore_axis_name="s",
                                    num_cores=1),
        out_type=jax.ShapeDtypeStruct((_OPAD, _DPAD), jnp.float32),
        scratch_types=[
            pltpu.VMEM((_SLICE,), jnp.int32),
            pltpu.VMEM((16,), jnp.int32),
            pltpu.VMEM((_SLICE,), jnp.int32),
            pltpu.VMEM((_SLICE,), jnp.int32),
            pltpu.VMEM((_SLICE,), jnp.int32),
            pltpu.VMEM((16,), jnp.int32),
            pltpu.VMEM((_LCAP,), jnp.int32),
            pltpu.VMEM((_LCAP,), jnp.int32),
            pltpu.VMEM((_ST * 16,), jnp.int32),
            pltpu.VMEM((16,), jnp.int32),
            pltpu.VMEM((16,), jnp.int32),
            pltpu.VMEM_SHARED((_ST * _SLICE,), jnp.int32),
            pltpu.VMEM_SHARED((_ST * _SLICE,), jnp.int32),
            pltpu.VMEM_SHARED((_ST * 16,), jnp.int32),
            pltpu.VMEM((16, _DPAD), jnp.float32),
            pltpu.SemaphoreType.DMA,
        ],
        compiler_params=pltpu.CompilerParams(needs_layout_passes=False),
    )


def kernel(src_pts, tgt_pts, W1, b1, W2, b2, W3, b3):
    keys2d, thr2d, tbl = _mlp_keys(src_pts, W1, b1, W2, b2, W3)
    keys = keys2d.reshape(N)
    thr = thr2d.reshape(128)
    out = _sc_topk_gather()(tbl, keys, thr)
    g = out[:NKEY, :_R]
    return jnp.transpose(g.reshape(NKEY, B, C), (1, 0, 2))
